# baseline (speedup 1.0000x reference)
import functools

import jax
import jax.numpy as jnp
from jax import lax
from jax.experimental import pallas as pl
from jax.experimental.pallas import tpu as pltpu
from jax.experimental.pallas import tpu_sc as plsc

B = 16384
D = 64
NC = 2
NS = 16
NW = NC * NS
BPW = B // NW
CH = 128
NCH = BPW // CH

_mesh = plsc.VectorSubcoreMesh(core_axis_name="c", subcore_axis_name="s")


@functools.partial(
    pl.kernel,
    mesh=_mesh,
    out_type=jax.ShapeDtypeStruct((B,), jnp.float32),
    compiler_params=pltpu.CompilerParams(
        needs_layout_passes=False, use_tc_tiling_on_sc=False
    ),
    scratch_types=[
        pltpu.VMEM((NCH, CH), jnp.int32),
        pltpu.VMEM((NCH, CH), jnp.int32),
        pltpu.VMEM((NCH, CH), jnp.int32),
        pltpu.VMEM((BPW, D), jnp.float32),
        pltpu.VMEM((BPW, D), jnp.float32),
        pltpu.VMEM((BPW, D), jnp.float32),
        pltpu.VMEM((BPW,), jnp.float32),
        pltpu.SemaphoreType.DMA,
    ],
)
def _transe_sc(sub_hbm, obj_hbm, rel_hbm, ent_hbm, relemb_hbm, out_hbm,
               sidx, oidx, ridx, srow, orow, rrow, outv, sem):
    wid = lax.axis_index("s") * NC + lax.axis_index("c")

    pltpu.sync_copy(sub_hbm.at[wid], sidx)
    pltpu.sync_copy(obj_hbm.at[wid], oidx)
    pltpu.sync_copy(rel_hbm.at[wid], ridx)

    copies = []
    for j in range(NCH):
        dst = pl.ds(j * CH, CH)
        copies.append(pltpu.async_copy(ent_hbm.at[sidx.at[j]], srow.at[dst], sem))
        copies.append(pltpu.async_copy(ent_hbm.at[oidx.at[j]], orow.at[dst], sem))
        copies.append(pltpu.async_copy(relemb_hbm.at[ridx.at[j]], rrow.at[dst], sem))
    for c in copies:
        c.wait()

    lane = lax.iota(jnp.int32, 16)

    def block(rb, carry):
        row_ids = rb * 16 + lane
        acc = jnp.zeros((16,), jnp.float32)
        for j in range(D):
            cj = jnp.full((16,), j, jnp.int32)
            s = plsc.load_gather(srow, [row_ids, cj])
            r = plsc.load_gather(rrow, [row_ids, cj])
            o = plsc.load_gather(orow, [row_ids, cj])
            d = s + r - o
            acc = acc + d * d
        outv[pl.ds(rb * 16, 16)] = acc
        return carry

    lax.fori_loop(0, BPW // 16, block, 0)
    pltpu.sync_copy(outv, out_hbm.at[pl.ds(wid * BPW, BPW)])


def kernel(subjects, objects, relations, ent_emb, rel_emb):
    sub = subjects.astype(jnp.int32).reshape(NW, NCH, CH)
    obj = objects.astype(jnp.int32).reshape(NW, NCH, CH)
    rel = relations.astype(jnp.int32).reshape(NW, NCH, CH)
    out = _transe_sc(sub, obj, rel, ent_emb, rel_emb)
    return out.reshape(-1, 1)
